# baseline (speedup 1.0000x reference)
import functools

import jax
import jax.numpy as jnp
from jax.experimental import pallas as pl
from jax.experimental.pallas import tpu as pltpu


def _se_kernel(x_ref, w1_ref, w2_ref, o_ref, *, inv_hw):
    xb = x_ref[0]
    pooled = jnp.sum(xb, axis=1, keepdims=True) * inv_hw
    h = jnp.maximum(jnp.sum(w1_ref[...] * pooled, axis=0, keepdims=True), 0.0)
    s = jax.nn.sigmoid(jnp.sum(w2_ref[...] * h, axis=1, keepdims=True))
    o_ref[0] = xb * s


def kernel(x, w1, w2):
    B, C, H, W = x.shape
    HW = H * W
    Cr = w1.shape[0]

    x3 = x.reshape(B, C, HW)
    w1t = w1.T.astype(jnp.float32)
    w2f = w2.astype(jnp.float32)

    body = functools.partial(_se_kernel, inv_hw=1.0 / float(HW))
    out3 = pl.pallas_call(
        body,
        out_shape=jax.ShapeDtypeStruct((B, C, HW), x.dtype),
        grid=(B,),
        in_specs=[
            pl.BlockSpec((1, C, HW), lambda b: (b, 0, 0)),
            pl.BlockSpec((C, Cr), lambda b: (0, 0)),
            pl.BlockSpec((C, Cr), lambda b: (0, 0)),
        ],
        out_specs=pl.BlockSpec((1, C, HW), lambda b: (b, 0, 0)),
        compiler_params=pltpu.CompilerParams(
            dimension_semantics=("parallel",),
        ),
    )(x3, w1t, w2f)
    return out3.reshape(B, C, H, W)
